# baseline (speedup 1.0000x reference)
import functools

import jax
import jax.numpy as jnp
from jax import lax
from jax.experimental import pallas as pl
from jax.experimental.pallas import tpu as pltpu
from jax.experimental.pallas import tpu_sc as plsc

NTA = 8192
D = 256
E = 8
L = 16
NW = 32
TPW = NTA // NW
BLK = 256
NBLK = 40
NSORT = NBLK * BLK


def _wid():
    return lax.axis_index("s") * 2 + lax.axis_index("c")


def _lane_iota():
    return lax.broadcasted_iota(jnp.int32, (L,), 0)


def _full(x, dtype=jnp.int32):
    return jax.lax.broadcast_in_dim(jnp.asarray(x, dtype), (L,), ())


@functools.cache
def _build_sc_kernels():
    mesh = plsc.VectorSubcoreMesh(core_axis_name="c", subcore_axis_name="s")

    @functools.partial(
        pl.kernel,
        out_type=jax.ShapeDtypeStruct((NBLK, L), jnp.int32),
        mesh=mesh,
        compiler_params=pltpu.CompilerParams(
            needs_layout_passes=False, skip_device_barrier=True),
        scratch_types=[pltpu.VMEM((TPW,), jnp.int32), pltpu.VMEM((L,), jnp.int32)],
    )
    def hist(sym_hbm, cnt_hbm, sym_v, row_v):
        wid = _wid()
        pltpu.sync_copy(sym_hbm.at[pl.ds(wid * TPW, TPW)], sym_v)
        iota = _lane_iota()
        acc = jnp.zeros((L,), jnp.int32)
        for j in range(TPW // L):
            s = sym_v[pl.ds(j * L, L)]
            for e in range(E):
                te = jnp.sum((s == e).astype(jnp.int32))
                acc = acc + jnp.where(iota == e, _full(te), _full(0))
        row_v[...] = acc
        pltpu.sync_copy(row_v, cnt_hbm.at[wid])
        @pl.when(wid < NBLK - NW)
        def _():
            row_v[...] = jnp.zeros((L,), jnp.int32)
            pltpu.sync_copy(row_v, cnt_hbm.at[NW + wid])

    @functools.partial(
        pl.kernel,
        out_type=(
            jax.ShapeDtypeStruct((NSORT, D), jnp.float32),
            jax.ShapeDtypeStruct((NTA,), jnp.int32),
            jax.ShapeDtypeStruct((3 * L,), jnp.int32),
        ),
        mesh=mesh,
        compiler_params=pltpu.CompilerParams(
            needs_layout_passes=False, skip_device_barrier=True),
        scratch_types=[
            pltpu.VMEM((TPW,), jnp.int32),
            pltpu.VMEM((NBLK, L), jnp.int32),
            pltpu.VMEM((L,), jnp.int32),
            pltpu.VMEM((2, 128), jnp.int32),
            pltpu.VMEM((2, 128, D), jnp.float32),
            pltpu.VMEM((3 * L,), jnp.int32),
            pltpu.SemaphoreType.DMA,
        ],
    )
    def route_scatter(sym_hbm, cnt_hbm, rho_hbm, rho_sorted_hbm, slot_hbm,
                      blkexp_hbm, sym_v, cnt_v, basebuf, pos2, rho_v, bex_v,
                      sem):
        wid = _wid()
        iota = _lane_iota()
        pltpu.sync_copy(sym_hbm.at[pl.ds(wid * TPW, TPW)], sym_v)
        pltpu.sync_copy(cnt_hbm, cnt_v)

        tot = jnp.zeros((L,), jnp.int32)
        pref = jnp.zeros((L,), jnp.int32)
        for w2 in range(NW):
            r = cnt_v[w2]
            tot = tot + r
            pref = pref + jnp.where(_full(w2, jnp.int32) < _full(wid), r,
                                    _full(0))
        nblk = (tot + _full(BLK - 1)) >> _full(8)
        incl = plsc.cumsum(nblk)
        excl = incl - nblk
        base = (excl << _full(8)) + pref

        for j in range(TPW // L):
            s = sym_v[pl.ds(j * L, L)]
            basebuf[...] = base
            bg = plsc.load_gather(basebuf, [s])
            pos = jnp.zeros((L,), jnp.int32)
            for e in range(E):
                m = s == _full(e)
                mi = m.astype(jnp.int32)
                ci = plsc.cumsum(mi)
                pos = jnp.where(m, bg + ci - _full(1), pos)
                te = jnp.sum(mi)
                base = base + jnp.where(iota == _full(e), _full(te), _full(0))
            pos2[j >> 3, pl.ds((j & 7) * L, L)] = pos

        for k in range(2):
            pltpu.sync_copy(pos2.at[k],
                            slot_hbm.at[pl.ds(wid * TPW + k * 128, 128)])
            pltpu.sync_copy(rho_hbm.at[pl.ds(wid * TPW + k * 128, 128)],
                            rho_v.at[k])
        cps = [pltpu.async_copy(rho_v.at[k], rho_sorted_hbm.at[pos2.at[k]], sem)
               for k in range(2)]
        for cp in cps:
            cp.wait()

        @pl.when(wid == 0)
        def _():
            for kk in range(3):
                jv = iota + _full(kk * L)
                be = jnp.zeros((L,), jnp.int32)
                for e in range(E):
                    lo = jnp.sum(jnp.where(iota == _full(e), excl, _full(0)))
                    hi = jnp.sum(jnp.where(iota == _full(e), incl, _full(0)))
                    be = jnp.where((jv >= _full(lo)) & (jv < _full(hi)),
                                   _full(e), be)
                bex_v[pl.ds(kk * L, L)] = be
            pltpu.sync_copy(bex_v, blkexp_hbm)

    @functools.partial(
        pl.kernel,
        out_type=jax.ShapeDtypeStruct((NTA, D), jnp.float32),
        mesh=mesh,
        compiler_params=pltpu.CompilerParams(
            needs_layout_passes=False, skip_device_barrier=True),
        scratch_types=[
            pltpu.VMEM((2, 128), jnp.int32),
            pltpu.VMEM((2, 128, D), jnp.float32),
            pltpu.SemaphoreType.DMA,
        ],
    )
    def unsort(slot_hbm, y_hbm, coeff_hbm, pos2, y_v, sem):
        wid = _wid()
        for k in range(2):
            pltpu.sync_copy(slot_hbm.at[pl.ds(wid * TPW + k * 128, 128)],
                            pos2.at[k])
        cps = [pltpu.async_copy(y_hbm.at[pos2.at[k]], y_v.at[k], sem)
               for k in range(2)]
        for cp in cps:
            cp.wait()
        for k in range(2):
            pltpu.sync_copy(y_v.at[k],
                            coeff_hbm.at[pl.ds(wid * TPW + k * 128, 128)])

    return hist, route_scatter, unsort


def _gmm_body(be_ref, x_ref, w_ref, b_ref, o_ref):
    x = x_ref[...].astype(jnp.bfloat16)
    o_ref[...] = (
        jnp.dot(x, w_ref[0], preferred_element_type=jnp.float32) + b_ref[0]
    )


def _grouped_matmul(blkexp, rho_sorted, w_bf, b):
    grid_spec = pltpu.PrefetchScalarGridSpec(
        num_scalar_prefetch=1,
        grid=(NBLK,),
        in_specs=[
            pl.BlockSpec((BLK, D), lambda i, be: (i, 0)),
            pl.BlockSpec((1, D, D), lambda i, be: (be[i], 0, 0)),
            pl.BlockSpec((1, 1, D), lambda i, be: (be[i], 0, 0)),
        ],
        out_specs=pl.BlockSpec((BLK, D), lambda i, be: (i, 0)),
    )
    return pl.pallas_call(
        _gmm_body,
        grid_spec=grid_spec,
        out_shape=jax.ShapeDtypeStruct((NSORT, D), jnp.float32),
    )(blkexp, rho_sorted, w_bf, b.reshape(E, 1, D))


def kernel(rho, symbols, W, b):
    hist, route_scatter, unsort = _build_sc_kernels()
    cnt = hist(symbols)
    rho_sorted, slot, blkexp = route_scatter(symbols, cnt, rho)
    y = _grouped_matmul(blkexp[:NBLK], rho_sorted, W.astype(jnp.bfloat16), b)
    return unsort(slot, y)


# speedup vs baseline: 1.0299x; 1.0299x over previous
import functools

import jax
import jax.numpy as jnp
from jax import lax
from jax.experimental import pallas as pl
from jax.experimental.pallas import tpu as pltpu
from jax.experimental.pallas import tpu_sc as plsc

NTA = 8192
D = 256
E = 8
L = 16
NW = 32
TPW = NTA // NW
BLK = 256
NBLK = 40
NSORT = NBLK * BLK


def _wid():
    return lax.axis_index("s") * 2 + lax.axis_index("c")


def _lane_iota():
    return lax.broadcasted_iota(jnp.int32, (L,), 0)


def _full(x, dtype=jnp.int32):
    return jax.lax.broadcast_in_dim(jnp.asarray(x, dtype), (L,), ())


@functools.cache
def _build_sc_kernels():
    mesh = plsc.VectorSubcoreMesh(core_axis_name="c", subcore_axis_name="s")

    @functools.partial(
        pl.kernel,
        out_type=jax.ShapeDtypeStruct((NBLK, L), jnp.int32),
        mesh=mesh,
        compiler_params=pltpu.CompilerParams(
            needs_layout_passes=False, skip_device_barrier=True),
        scratch_types=[pltpu.VMEM((TPW,), jnp.int32), pltpu.VMEM((L,), jnp.int32)],
    )
    def hist(sym_hbm, cnt_hbm, sym_v, row_v):
        wid = _wid()
        pltpu.sync_copy(sym_hbm.at[pl.ds(wid * TPW, TPW)], sym_v)
        iota = _lane_iota()
        acc = jnp.zeros((L,), jnp.int32)
        for j in range(TPW // L):
            s = sym_v[pl.ds(j * L, L)]
            for e in range(E):
                te = jnp.sum((s == e).astype(jnp.int32))
                acc = acc + jnp.where(iota == e, _full(te), _full(0))
        row_v[...] = acc
        pltpu.sync_copy(row_v, cnt_hbm.at[wid])
        @pl.when(wid < NBLK - NW)
        def _():
            row_v[...] = jnp.zeros((L,), jnp.int32)
            pltpu.sync_copy(row_v, cnt_hbm.at[NW + wid])

    @functools.partial(
        pl.kernel,
        out_type=(
            jax.ShapeDtypeStruct((NSORT, D), jnp.float32),
            jax.ShapeDtypeStruct((NTA,), jnp.int32),
            jax.ShapeDtypeStruct((3 * L,), jnp.int32),
        ),
        mesh=mesh,
        compiler_params=pltpu.CompilerParams(
            needs_layout_passes=False, skip_device_barrier=True),
        scratch_types=[
            pltpu.VMEM((TPW,), jnp.int32),
            pltpu.VMEM((NBLK, L), jnp.int32),
            pltpu.VMEM((L,), jnp.int32),
            pltpu.VMEM((2, 128), jnp.int32),
            pltpu.VMEM((2, 128, D), jnp.float32),
            pltpu.VMEM((3 * L,), jnp.int32),
            pltpu.SemaphoreType.DMA,
            pltpu.SemaphoreType.DMA,
        ],
    )
    def route_scatter(sym_hbm, cnt_hbm, rho_hbm, rho_sorted_hbm, slot_hbm,
                      blkexp_hbm, sym_v, cnt_v, basebuf, pos2, rho_v, bex_v,
                      sem, sem2):
        wid = _wid()
        iota = _lane_iota()
        pltpu.sync_copy(sym_hbm.at[pl.ds(wid * TPW, TPW)], sym_v)
        rho_cps = [
            pltpu.async_copy(rho_hbm.at[pl.ds(wid * TPW + k * 128, 128)],
                             rho_v.at[k], sem2)
            for k in range(2)
        ]
        pltpu.sync_copy(cnt_hbm, cnt_v)

        tot = jnp.zeros((L,), jnp.int32)
        pref = jnp.zeros((L,), jnp.int32)
        for w2 in range(NW):
            r = cnt_v[w2]
            tot = tot + r
            pref = pref + jnp.where(_full(w2, jnp.int32) < _full(wid), r,
                                    _full(0))
        nblk = (tot + _full(BLK - 1)) >> _full(8)
        incl = plsc.cumsum(nblk)
        excl = incl - nblk
        base = (excl << _full(8)) + pref

        for j in range(TPW // L):
            s = sym_v[pl.ds(j * L, L)]
            basebuf[...] = base
            bg = plsc.load_gather(basebuf, [s])
            pos = jnp.zeros((L,), jnp.int32)
            for e in range(E):
                m = s == _full(e)
                mi = m.astype(jnp.int32)
                ci = plsc.cumsum(mi)
                pos = jnp.where(m, bg + ci - _full(1), pos)
                te = jnp.sum(mi)
                base = base + jnp.where(iota == _full(e), _full(te), _full(0))
            pos2[j >> 3, pl.ds((j & 7) * L, L)] = pos

        for k in range(2):
            pltpu.sync_copy(pos2.at[k],
                            slot_hbm.at[pl.ds(wid * TPW + k * 128, 128)])
        for cp in rho_cps:
            cp.wait()
        cps = [pltpu.async_copy(rho_v.at[k], rho_sorted_hbm.at[pos2.at[k]], sem)
               for k in range(2)]
        for cp in cps:
            cp.wait()

        @pl.when(wid == 0)
        def _():
            for kk in range(3):
                jv = iota + _full(kk * L)
                be = jnp.zeros((L,), jnp.int32)
                for e in range(E):
                    lo = jnp.sum(jnp.where(iota == _full(e), excl, _full(0)))
                    hi = jnp.sum(jnp.where(iota == _full(e), incl, _full(0)))
                    be = jnp.where((jv >= _full(lo)) & (jv < _full(hi)),
                                   _full(e), be)
                bex_v[pl.ds(kk * L, L)] = be
            pltpu.sync_copy(bex_v, blkexp_hbm)

    @functools.partial(
        pl.kernel,
        out_type=jax.ShapeDtypeStruct((NTA, D), jnp.float32),
        mesh=mesh,
        compiler_params=pltpu.CompilerParams(
            needs_layout_passes=False, skip_device_barrier=True),
        scratch_types=[
            pltpu.VMEM((2, 128), jnp.int32),
            pltpu.VMEM((2, 128, D), jnp.float32),
            pltpu.SemaphoreType.DMA,
        ],
    )
    def unsort(slot_hbm, y_hbm, coeff_hbm, pos2, y_v, sem):
        wid = _wid()
        for k in range(2):
            pltpu.sync_copy(slot_hbm.at[pl.ds(wid * TPW + k * 128, 128)],
                            pos2.at[k])
        cps = [pltpu.async_copy(y_hbm.at[pos2.at[k]], y_v.at[k], sem)
               for k in range(2)]
        for cp in cps:
            cp.wait()
        for k in range(2):
            pltpu.sync_copy(y_v.at[k],
                            coeff_hbm.at[pl.ds(wid * TPW + k * 128, 128)])

    return hist, route_scatter, unsort


def _gmm_body(be_ref, x_ref, w_ref, b_ref, o_ref):
    be = be_ref[pl.program_id(0)]
    x = x_ref[...].astype(jnp.bfloat16)
    o_ref[...] = (
        jnp.dot(x, w_ref[be], preferred_element_type=jnp.float32) + b_ref[be]
    )


def _grouped_matmul(blkexp, rho_sorted, w_bf, b):
    grid_spec = pltpu.PrefetchScalarGridSpec(
        num_scalar_prefetch=1,
        grid=(NBLK,),
        in_specs=[
            pl.BlockSpec((BLK, D), lambda i, be: (i, 0)),
            pl.BlockSpec((E, D, D), lambda i, be: (0, 0, 0)),
            pl.BlockSpec((E, 1, D), lambda i, be: (0, 0, 0)),
        ],
        out_specs=pl.BlockSpec((BLK, D), lambda i, be: (i, 0)),
    )
    return pl.pallas_call(
        _gmm_body,
        grid_spec=grid_spec,
        out_shape=jax.ShapeDtypeStruct((NSORT, D), jnp.float32),
    )(blkexp, rho_sorted, w_bf, b.reshape(E, 1, D))


def kernel(rho, symbols, W, b):
    hist, route_scatter, unsort = _build_sc_kernels()
    cnt = hist(symbols)
    rho_sorted, slot, blkexp = route_scatter(symbols, cnt, rho)
    y = _grouped_matmul(blkexp[:NBLK], rho_sorted, W.astype(jnp.bfloat16), b)
    return unsort(slot, y)


# speedup vs baseline: 2.3822x; 2.3131x over previous
import jax
import jax.numpy as jnp
from jax.experimental import pallas as pl

_NTA = 8192
_D = 256
_E = 8
_BLK = 1024


def _fused_masked_kernel(sym_ref, rho_ref, w_ref, b_ref, out_ref):
    sym = sym_ref[...]
    x = rho_ref[...]
    onehot = (sym == jax.lax.broadcasted_iota(jnp.int32, (_BLK, _E), 1))
    acc = jnp.dot(onehot.astype(jnp.bfloat16), b_ref[...],
                  preferred_element_type=jnp.float32)
    for e in range(_E):
        m = (sym == e)
        xm = jnp.where(m, x, jnp.bfloat16(0))
        acc += jnp.dot(xm, w_ref[e], preferred_element_type=jnp.float32)
    out_ref[...] = acc


def kernel(rho, symbols, W, b):
    sym2d = symbols.reshape(_NTA, 1)
    rho_bf = rho.astype(jnp.bfloat16)
    w_bf = W.astype(jnp.bfloat16)
    b_bf = b.astype(jnp.bfloat16)
    grid = _NTA // _BLK
    return pl.pallas_call(
        _fused_masked_kernel,
        grid=(grid,),
        in_specs=[
            pl.BlockSpec((_BLK, 1), lambda i: (i, 0)),
            pl.BlockSpec((_BLK, _D), lambda i: (i, 0)),
            pl.BlockSpec((_E, _D, _D), lambda i: (0, 0, 0)),
            pl.BlockSpec((_E, _D), lambda i: (0, 0)),
        ],
        out_specs=pl.BlockSpec((_BLK, _D), lambda i: (i, 0)),
        out_shape=jax.ShapeDtypeStruct((_NTA, _D), jnp.float32),
    )(sym2d, rho_bf, w_bf, b_bf)
